# baseline (speedup 1.0000x reference)
import functools

import jax
import jax.numpy as jnp
from jax import lax
from jax.experimental import pallas as pl
from jax.experimental.pallas import tpu as pltpu
from jax.experimental.pallas import tpu_sc as plsc

_NC = 2
_NS = 16
_NW = _NC * _NS

_BLK = 128
_L = 16


@functools.cache
def _make_gather(F, B, V4, D):
    n_blocks = B // (_NW * _BLK)
    assert D % _L == 0 and B % (_NW * _BLK) == 0 and F % 2 == 0

    mesh = plsc.VectorSubcoreMesh(
        core_axis_name="c", subcore_axis_name="s",
        num_cores=_NC, num_subcores=_NS)

    def body(xT_hbm, w_hbm, out_hbm,
             idx0, idx1, g0, g1, buf0, buf1, stage, sem):
        wid = lax.axis_index("s") * _NC + lax.axis_index("c")

        iota = lax.iota(jnp.int32, _L)
        rows = [iota + k * _L for k in range(_BLK // _L)]

        def prep(f, b0, g_ref, idx_ref, buf_ref):
            pltpu.sync_copy(xT_hbm.at[f, pl.ds(b0, _BLK)], idx_ref)
            for k in range(_BLK // _L):
                sl = pl.ds(k * _L, _L)
                g_ref[sl] = lax.shift_right_logical(idx_ref[sl], 2)
            pltpu.async_copy(w_hbm.at[g_ref], buf_ref, sem)

        def drain(g_ref, buf_ref):
            pltpu.make_async_copy(w_hbm.at[g_ref], buf_ref, sem).wait()

        def extract(f, b0, idx_ref, buf_ref):
            cols = []
            for k in range(_BLK // _L):
                q = lax.bitwise_and(idx_ref[pl.ds(k * _L, _L)], 3)
                cols.append(lax.mul(q, D))
            for d in range(D):
                for k in range(_BLK // _L):
                    vals = plsc.load_gather(buf_ref, [rows[k], cols[k] + d])
                    stage[d, pl.ds(k * _L, _L)] = vals
            pltpu.sync_copy(stage, out_hbm.at[f, :, pl.ds(b0, _BLK)])

        def block(b, carry):
            b0 = (wid * n_blocks + b) * _BLK
            prep(0, b0, g0, idx0, buf0)

            def step(i, carry2):
                f = i * 2
                prep(f + 1, b0, g1, idx1, buf1)
                drain(g0, buf0)
                extract(f, b0, idx0, buf0)
                prep(f + 2, b0, g0, idx0, buf0)
                drain(g1, buf1)
                extract(f + 1, b0, idx1, buf1)
                return carry2

            lax.fori_loop(0, F // 2 - 1, step, 0)
            prep(F - 1, b0, g1, idx1, buf1)
            drain(g0, buf0)
            extract(F - 2, b0, idx0, buf0)
            drain(g1, buf1)
            extract(F - 1, b0, idx1, buf1)
            return carry

        lax.fori_loop(0, n_blocks, block, 0)

    return pl.kernel(
        body,
        out_type=jax.ShapeDtypeStruct((F, D, B), jnp.float32),
        mesh=mesh,
        compiler_params=pltpu.CompilerParams(use_tc_tiling_on_sc=True, needs_layout_passes=False),
        scratch_types=[
            pltpu.VMEM((_BLK,), jnp.int32),
            pltpu.VMEM((_BLK,), jnp.int32),
            pltpu.VMEM((_BLK,), jnp.int32),
            pltpu.VMEM((_BLK,), jnp.int32),
            pltpu.VMEM((_BLK, 4 * D), jnp.float32),
            pltpu.VMEM((_BLK, 4 * D), jnp.float32),
            pltpu.VMEM((D, _BLK), jnp.float32),
            pltpu.SemaphoreType.DMA,
        ],
    )


def kernel(x, weight):
    batch, n_fields = x.shape
    v, d = weight.shape
    xT = x.T
    w4 = weight.reshape(v // 4, 4 * d)
    out = _make_gather(n_fields, batch, v // 4, d)(xT, w4)
    return jnp.transpose(out, (2, 0, 1))
